# baseline (speedup 1.0000x reference)
import functools

import jax
import jax.numpy as jnp
from jax import lax
from jax.experimental import pallas as pl
from jax.experimental.pallas import tpu as pltpu
from jax.experimental.pallas import tpu_sc as plsc

_NC = 2
_NS = 16
_NW = _NC * _NS
_L = 16


@functools.lru_cache(maxsize=None)
def _make_sc_add(total_rows: int, table_rows: int, d: int):
    rows_per_w = total_rows // _NW
    chunk_rows = 32
    C = chunk_rows * d
    n_chunks = rows_per_w // chunk_rows
    mesh = plsc.VectorSubcoreMesh(
        core_axis_name="c", subcore_axis_name="s",
        num_cores=_NC, num_subcores=_NS,
    )

    @functools.partial(
        pl.kernel,
        out_type=jax.ShapeDtypeStruct((total_rows * d,), jnp.float32),
        mesh=mesh,
        scratch_types=[
            pltpu.VMEM((C,), jnp.float32),
            pltpu.VMEM((C,), jnp.float32),
        ],
    )
    def sc_add(x_hbm, p_hbm, o_hbm, xbuf, pbuf):
        wid = lax.axis_index("s") * _NC + lax.axis_index("c")
        x_base = wid * (rows_per_w * d)
        p_base = (wid * rows_per_w % table_rows) * d

        def chunk_body(ci, carry):
            off = ci * C
            pltpu.sync_copy(x_hbm.at[pl.ds(x_base + off, C)], xbuf)
            pltpu.sync_copy(p_hbm.at[pl.ds(p_base + off, C)], pbuf)

            @plsc.parallel_loop(0, C // _L, unroll=8)
            def add_body(i):
                sl = pl.ds(i * _L, _L)
                xbuf[sl] = xbuf[sl] + pbuf[sl]

            pltpu.sync_copy(xbuf, o_hbm.at[pl.ds(x_base + off, C)])
            return carry

        lax.fori_loop(0, n_chunks, chunk_body, 0)

    return sc_add


def kernel(x, pos_table):
    B, S, D = x.shape
    fn = _make_sc_add(B * S, S, D)
    out = fn(x.reshape(-1), pos_table[:S].reshape(-1))
    return out.reshape(x.shape)


# speedup vs baseline: 1.2355x; 1.2355x over previous
import functools

import jax
import jax.numpy as jnp
from jax import lax
from jax.experimental import pallas as pl
from jax.experimental.pallas import tpu as pltpu
from jax.experimental.pallas import tpu_sc as plsc

_NC = 2
_NS = 16
_NW = _NC * _NS
_L = 16


@functools.lru_cache(maxsize=None)
def _make_sc_add(total_rows: int, table_rows: int, d: int):
    rows_per_w = total_rows // _NW
    chunk_rows = 16
    C = chunk_rows * d
    n_chunks = rows_per_w // chunk_rows
    assert n_chunks % 2 == 0 and n_chunks >= 4
    mesh = plsc.VectorSubcoreMesh(
        core_axis_name="c", subcore_axis_name="s",
        num_cores=_NC, num_subcores=_NS,
    )

    @functools.partial(
        pl.kernel,
        out_type=jax.ShapeDtypeStruct((total_rows * d,), jnp.float32),
        mesh=mesh,
        scratch_types=(
            [pltpu.VMEM((C,), jnp.float32)] * 6
            + [pltpu.SemaphoreType.DMA] * 6
        ),
    )
    def sc_add(x_hbm, p_hbm, o_hbm,
               xb0, xb1, pb0, pb1, ob0, ob1,
               sx0, sx1, sp0, sp1, so0, so1):
        xb, pb, ob = [xb0, xb1], [pb0, pb1], [ob0, ob1]
        sx, sp, so = [sx0, sx1], [sp0, sp1], [so0, so1]

        wid = lax.axis_index("s") * _NC + lax.axis_index("c")
        x_base = wid * (rows_per_w * d)
        p_base = (wid * rows_per_w % table_rows) * d

        def start_in(c, b):
            off = c * C
            pltpu.make_async_copy(
                x_hbm.at[pl.ds(x_base + off, C)], xb[b], sx[b]).start()
            pltpu.make_async_copy(
                p_hbm.at[pl.ds(p_base + off, C)], pb[b], sp[b]).start()

        for b in range(2):
            start_in(b, b)

        def pair_body(g, carry):
            for b in range(2):
                c = g * 2 + b
                pltpu.make_async_copy(
                    x_hbm.at[pl.ds(x_base, C)], xb[b], sx[b]).wait()
                pltpu.make_async_copy(
                    p_hbm.at[pl.ds(p_base, C)], pb[b], sp[b]).wait()

                @pl.when(g >= 1)
                def _():
                    pltpu.make_async_copy(
                        ob[b], o_hbm.at[pl.ds(x_base, C)], so[b]).wait()

                @plsc.parallel_loop(0, C // _L, unroll=8)
                def _(i):
                    sl = pl.ds(i * _L, _L)
                    ob[b][sl] = xb[b][sl] + pb[b][sl]

                pltpu.make_async_copy(
                    ob[b], o_hbm.at[pl.ds(x_base + c * C, C)], so[b]).start()

                @pl.when(c + 2 < n_chunks)
                def _():
                    start_in(c + 2, b)
            return carry

        lax.fori_loop(0, n_chunks // 2, pair_body, 0)

        for b in range(2):
            pltpu.make_async_copy(
                ob[b], o_hbm.at[pl.ds(x_base, C)], so[b]).wait()

    return sc_add


def kernel(x, pos_table):
    B, S, D = x.shape
    fn = _make_sc_add(B * S, S, D)
    out = fn(x.reshape(-1), pos_table[:S].reshape(-1))
    return out.reshape(x.shape)


# speedup vs baseline: 1.3962x; 1.1301x over previous
import functools

import jax
import jax.numpy as jnp
from jax import lax
from jax.experimental import pallas as pl
from jax.experimental.pallas import tpu as pltpu
from jax.experimental.pallas import tpu_sc as plsc

_NC = 2
_NS = 16
_NW = _NC * _NS
_L = 16


@functools.lru_cache(maxsize=None)
def _make_sc_add(sc_rows: int, table_rows: int, d: int):
    rows_per_w = sc_rows // _NW
    chunk_rows = 16
    C = chunk_rows * d
    n_chunks = rows_per_w // chunk_rows
    assert n_chunks % 2 == 0 and n_chunks >= 4
    mesh = plsc.VectorSubcoreMesh(
        core_axis_name="c", subcore_axis_name="s",
        num_cores=_NC, num_subcores=_NS,
    )

    @functools.partial(
        pl.kernel,
        out_type=jax.ShapeDtypeStruct((sc_rows * d,), jnp.float32),
        mesh=mesh,
        scratch_types=(
            [pltpu.VMEM((C,), jnp.float32)] * 6
            + [pltpu.SemaphoreType.DMA] * 6
        ),
    )
    def sc_add(x_hbm, p_hbm, o_hbm,
               xb0, xb1, pb0, pb1, ob0, ob1,
               sx0, sx1, sp0, sp1, so0, so1):
        xb, pb, ob = [xb0, xb1], [pb0, pb1], [ob0, ob1]
        sx, sp, so = [sx0, sx1], [sp0, sp1], [so0, so1]

        wid = lax.axis_index("s") * _NC + lax.axis_index("c")
        x_base = wid * (rows_per_w * d)
        p_base = (wid * rows_per_w % table_rows) * d

        def start_in(c, b):
            off = c * C
            pltpu.make_async_copy(
                x_hbm.at[pl.ds(x_base + off, C)], xb[b], sx[b]).start()
            pltpu.make_async_copy(
                p_hbm.at[pl.ds(p_base + off, C)], pb[b], sp[b]).start()

        for b in range(2):
            start_in(b, b)

        def pair_body(g, carry):
            for b in range(2):
                c = g * 2 + b
                pltpu.make_async_copy(
                    x_hbm.at[pl.ds(x_base, C)], xb[b], sx[b]).wait()
                pltpu.make_async_copy(
                    p_hbm.at[pl.ds(p_base, C)], pb[b], sp[b]).wait()

                @pl.when(g >= 1)
                def _():
                    pltpu.make_async_copy(
                        ob[b], o_hbm.at[pl.ds(x_base, C)], so[b]).wait()

                @plsc.parallel_loop(0, C // _L, unroll=8)
                def _(i):
                    sl = pl.ds(i * _L, _L)
                    ob[b][sl] = xb[b][sl] + pb[b][sl]

                pltpu.make_async_copy(
                    ob[b], o_hbm.at[pl.ds(x_base + c * C, C)], so[b]).start()

                @pl.when(c + 2 < n_chunks)
                def _():
                    start_in(c + 2, b)
            return carry

        lax.fori_loop(0, n_chunks // 2, pair_body, 0)

        for b in range(2):
            pltpu.make_async_copy(
                ob[b], o_hbm.at[pl.ds(x_base, C)], so[b]).wait()

    return sc_add


def _tc_add_body(x_ref, p_ref, o_ref):
    o_ref[...] = x_ref[...] + p_ref[...]


def _tc_add(x, pos, b_lo, bs):
    B, S, D = x.shape
    nb = B - b_lo
    grid = (S // bs, nb)
    return pl.pallas_call(
        _tc_add_body,
        grid=grid,
        in_specs=[
            pl.BlockSpec((1, bs, D), lambda s, b: (b + b_lo, s, 0)),
            pl.BlockSpec((bs, D), lambda s, b: (s, 0)),
        ],
        out_specs=pl.BlockSpec((1, bs, D), lambda s, b: (b, s, 0)),
        out_shape=jax.ShapeDtypeStruct((nb, S, D), x.dtype),
    )(x, pos)


def kernel(x, pos_table):
    B, S, D = x.shape
    pos = pos_table[:S]
    sc_batches = 1
    sc_fn = _make_sc_add(sc_batches * S, S, D)
    sc_out = sc_fn(x.reshape(-1), pos.reshape(-1))
    tc_out = _tc_add(x, pos, sc_batches, 512)
    return jnp.concatenate(
        [sc_out.reshape(sc_batches, S, D), tc_out], axis=0)
